# baseline (speedup 1.0000x reference)
import functools

import jax
import jax.numpy as jnp
from jax import lax
from jax.experimental import pallas as pl
from jax.experimental.pallas import tpu as pltpu
from jax.experimental.pallas import tpu_sc as plsc

N_NODES = 10000
N_EDGES = 320000
D_FEAT = 128

C = 128
NCHUNK = N_EDGES // C
NW = 32
MAXC = (NCHUNK + NW - 1) // NW
LANES = 16
WPE = D_FEAT // 2 // LANES
NBUF = 3
STG = 32

_GATHER_DNUMS = lax.GatherDimensionNumbers(
    offset_dims=(), collapsed_slice_dims=(0,), start_index_map=(0,))


def _permute16(x, perm):
    return lax.gather(x, perm.reshape(LANES, 1), _GATHER_DNUMS, (1,),
                      mode=lax.GatherScatterMode.PROMISE_IN_BOUNDS)


def _dot_chunk(rows_s, rows_d, scores_v, out_base):
    lane_iota = lax.iota(jnp.int32, LANES)
    perms = [jnp.bitwise_xor(lane_iota, sh) for sh in (8, 4, 2, 1)]
    lane_masks = [lane_iota == j for j in range(LANES)]

    def group(g, carry):
        e0 = g * LANES
        acc = jnp.zeros((LANES,), jnp.float32)
        for j in range(LANES):
            e = e0 + j
            p0 = jnp.zeros((LANES,), jnp.float32)
            p1 = jnp.zeros((LANES,), jnp.float32)
            for k in range(WPE):
                vs = rows_s[e, pl.ds(k * LANES, LANES)]
                vd = rows_d[e, pl.ds(k * LANES, LANES)]
                sa = lax.bitcast_convert_type(vs << 16, jnp.float32)
                sb = lax.bitcast_convert_type(vs, jnp.float32)
                da = lax.bitcast_convert_type(vd << 16, jnp.float32)
                db = lax.bitcast_convert_type(vd, jnp.float32)
                p0 = p0 + sa * da
                p1 = p1 + sb * db
            p = p0 + p1
            for perm in perms:
                p = p + _permute16(p, perm)
            acc = jnp.where(lane_masks[j], p, acc)
        scores_v[pl.ds(out_base + e0, LANES)] = acc
        return carry

    lax.fori_loop(0, C // LANES, group, 0)


def _sc_kernel(h_hbm, ei_hbm, out_hbm,
               table_sh, f32stage, pkstage, idx_s, idx_d,
               rows_s0, rows_s1, rows_s2,
               rows_d0, rows_d1, rows_d2, scores_v,
               sem_s0, sem_s1, sem_s2, sem_d0, sem_d1, sem_d2):
    info = plsc.get_sparse_core_info()
    nw = info.num_cores * info.num_subcores
    sid = lax.axis_index("s")
    wid = sid * info.num_cores + lax.axis_index("c")
    start = (wid * NCHUNK) // nw
    n_w = ((wid + 1) * NCHUNK) // nw - start

    rows_s = (rows_s0, rows_s1, rows_s2)
    rows_d = (rows_d0, rows_d1, rows_d2)
    sem_s = (sem_s0, sem_s1, sem_s2)
    sem_d = (sem_d0, sem_d1, sem_d2)

    rows_per_sub = N_NODES // 16
    base_row = sid * rows_per_sub
    c7fff = jnp.uint32(0x7FFF)
    c1 = jnp.uint32(1)
    cmask = jnp.uint32(0xFFFF0000)

    def _rnd(x):
        return x + c7fff + ((x >> 16) & c1)

    n_blk = (rows_per_sub + STG - 1) // STG
    for blk in range(n_blk):
        nrows = min(STG, rows_per_sub - blk * STG)
        r0 = base_row + blk * STG
        pltpu.sync_copy(h_hbm.at[pl.ds(r0, nrows)],
                        f32stage.at[pl.ds(0, nrows)])

        def pack_row(r, carry):
            for k in range(WPE):
                a = lax.bitcast_convert_type(
                    f32stage[r, pl.ds(k * LANES, LANES)], jnp.uint32)
                b = lax.bitcast_convert_type(
                    f32stage[r, pl.ds(D_FEAT // 2 + k * LANES, LANES)],
                    jnp.uint32)
                w = (_rnd(a) >> 16) | (_rnd(b) & cmask)
                pkstage[r, pl.ds(k * LANES, LANES)] = (
                    lax.bitcast_convert_type(w, jnp.int32))
            return carry

        lax.fori_loop(0, nrows, pack_row, 0)
        pltpu.sync_copy(pkstage.at[pl.ds(0, nrows)],
                        table_sh.at[pl.ds(r0, nrows)])

    pltpu.sync_copy(ei_hbm.at[0, pl.ds(start * C, MAXC * C)], idx_s)
    pltpu.sync_copy(ei_hbm.at[1, pl.ds(start * C, MAXC * C)], idx_d)

    plsc.subcore_barrier()

    def gather_pair(i, b):
        return (pltpu.make_async_copy(
                    table_sh.at[idx_s.at[pl.ds(i * C, C)]],
                    rows_s[b], sem_s[b]),
                pltpu.make_async_copy(
                    table_sh.at[idx_d.at[pl.ds(i * C, C)]],
                    rows_d[b], sem_d[b]))

    def issue(i, b):
        @pl.when(i < n_w)
        def _():
            cs, cd = gather_pair(i, b)
            cs.start()
            cd.start()

    for b in range(NBUF):
        issue(b, b)

    def body(t, carry):
        for b in range(NBUF):
            i = NBUF * t + b

            @pl.when(i < n_w)
            def _():
                cs, cd = gather_pair(i, b)
                cs.wait()
                cd.wait()
                _dot_chunk(rows_s[b], rows_d[b], scores_v, i * C)
                issue(i + NBUF, b)

        return carry

    lax.fori_loop(0, (MAXC + NBUF - 1) // NBUF, body, 0)

    pltpu.sync_copy(scores_v.at[pl.ds(0, (MAXC - 1) * C)],
                    out_hbm.at[pl.ds(start * C, (MAXC - 1) * C)])

    @pl.when(n_w == MAXC)
    def _():
        pltpu.sync_copy(scores_v.at[pl.ds((MAXC - 1) * C, C)],
                        out_hbm.at[pl.ds((start + MAXC - 1) * C, C)])


def kernel(h, edge_index):
    call = functools.partial(
        pl.kernel,
        out_type=jax.ShapeDtypeStruct((N_EDGES,), jnp.float32),
        mesh=plsc.VectorSubcoreMesh(core_axis_name="c", subcore_axis_name="s"),
        compiler_params=pltpu.CompilerParams(use_tc_tiling_on_sc=False,
                                             needs_layout_passes=False),
        scratch_types=[
            pltpu.VMEM_SHARED((N_NODES, D_FEAT // 2), jnp.int32),
            pltpu.VMEM((STG, D_FEAT), jnp.float32),
            pltpu.VMEM((STG, D_FEAT // 2), jnp.int32),
            pltpu.VMEM((MAXC * C,), jnp.int32),
            pltpu.VMEM((MAXC * C,), jnp.int32),
            pltpu.VMEM((C, D_FEAT // 2), jnp.int32),
            pltpu.VMEM((C, D_FEAT // 2), jnp.int32),
            pltpu.VMEM((C, D_FEAT // 2), jnp.int32),
            pltpu.VMEM((C, D_FEAT // 2), jnp.int32),
            pltpu.VMEM((C, D_FEAT // 2), jnp.int32),
            pltpu.VMEM((C, D_FEAT // 2), jnp.int32),
            pltpu.VMEM((MAXC * C,), jnp.float32),
            pltpu.SemaphoreType.DMA,
            pltpu.SemaphoreType.DMA,
            pltpu.SemaphoreType.DMA,
            pltpu.SemaphoreType.DMA,
            pltpu.SemaphoreType.DMA,
            pltpu.SemaphoreType.DMA,
        ],
    )(_sc_kernel)
    scores = call(h, edge_index)
    return scores.reshape(N_EDGES, 1)


# speedup vs baseline: 1.2470x; 1.2470x over previous
import functools

import jax
import jax.numpy as jnp
from jax import lax
from jax.experimental import pallas as pl
from jax.experimental.pallas import tpu as pltpu
from jax.experimental.pallas import tpu_sc as plsc

N_NODES = 10000
N_EDGES = 320000
D_FEAT = 128

C = 128
NCHUNK = N_EDGES // C
NW = 32
MAXC = (NCHUNK + NW - 1) // NW
LANES = 16
WPE = D_FEAT // 2 // LANES
NBUF = 3

_GATHER_DNUMS = lax.GatherDimensionNumbers(
    offset_dims=(), collapsed_slice_dims=(0,), start_index_map=(0,))


def _permute16(x, perm):
    return lax.gather(x, perm.reshape(LANES, 1), _GATHER_DNUMS, (1,),
                      mode=lax.GatherScatterMode.PROMISE_IN_BOUNDS)


def _dot_chunk(rows_s, rows_d, scores_v, out_base):
    lane_iota = lax.iota(jnp.int32, LANES)
    perms = [jnp.bitwise_xor(lane_iota, sh) for sh in (8, 4, 2, 1)]
    lane_masks = [lane_iota == j for j in range(LANES)]

    def group(g, carry):
        e0 = g * LANES
        acc = jnp.zeros((LANES,), jnp.float32)
        for j in range(LANES):
            e = e0 + j
            p0 = jnp.zeros((LANES,), jnp.float32)
            p1 = jnp.zeros((LANES,), jnp.float32)
            for k in range(WPE):
                vs = rows_s[e, pl.ds(k * LANES, LANES)]
                vd = rows_d[e, pl.ds(k * LANES, LANES)]
                sa = lax.bitcast_convert_type(vs << 16, jnp.float32)
                sb = lax.bitcast_convert_type(vs, jnp.float32)
                da = lax.bitcast_convert_type(vd << 16, jnp.float32)
                db = lax.bitcast_convert_type(vd, jnp.float32)
                p0 = p0 + sa * da
                p1 = p1 + sb * db
            p = p0 + p1
            for perm in perms:
                p = p + _permute16(p, perm)
            acc = jnp.where(lane_masks[j], p, acc)
        scores_v[pl.ds(out_base + e0, LANES)] = acc
        return carry

    lax.fori_loop(0, C // LANES, group, 0)


def _sc_kernel(h_hbm, ei_hbm, out_hbm,
               table_sh, idx_s, idx_d, rows_s0, rows_s1, rows_s2,
               rows_d0, rows_d1, rows_d2, scores_v,
               sem_s0, sem_s1, sem_s2, sem_d0, sem_d1, sem_d2):
    info = plsc.get_sparse_core_info()
    nw = info.num_cores * info.num_subcores
    sid = lax.axis_index("s")
    wid = sid * info.num_cores + lax.axis_index("c")
    start = (wid * NCHUNK) // nw
    n_w = ((wid + 1) * NCHUNK) // nw - start

    rows_s = (rows_s0, rows_s1, rows_s2)
    rows_d = (rows_d0, rows_d1, rows_d2)
    sem_s = (sem_s0, sem_s1, sem_s2)
    sem_d = (sem_d0, sem_d1, sem_d2)

    rows_per_sub = N_NODES // 16
    pltpu.sync_copy(h_hbm.at[pl.ds(sid * rows_per_sub, rows_per_sub)],
                    table_sh.at[pl.ds(sid * rows_per_sub, rows_per_sub)])

    pltpu.sync_copy(ei_hbm.at[0, pl.ds(start * C, MAXC * C)], idx_s)
    pltpu.sync_copy(ei_hbm.at[1, pl.ds(start * C, MAXC * C)], idx_d)

    plsc.subcore_barrier()

    def gather_pair(i, b):
        return (pltpu.make_async_copy(
                    table_sh.at[idx_s.at[pl.ds(i * C, C)]],
                    rows_s[b], sem_s[b]),
                pltpu.make_async_copy(
                    table_sh.at[idx_d.at[pl.ds(i * C, C)]],
                    rows_d[b], sem_d[b]))

    def issue(i, b):
        @pl.when(i < n_w)
        def _():
            cs, cd = gather_pair(i, b)
            cs.start()
            cd.start()

    for b in range(NBUF):
        issue(b, b)

    def body(t, carry):
        for b in range(NBUF):
            i = NBUF * t + b

            @pl.when(i < n_w)
            def _():
                cs, cd = gather_pair(i, b)
                cs.wait()
                cd.wait()
                _dot_chunk(rows_s[b], rows_d[b], scores_v, i * C)
                issue(i + NBUF, b)

        return carry

    lax.fori_loop(0, (MAXC + NBUF - 1) // NBUF, body, 0)

    pltpu.sync_copy(scores_v.at[pl.ds(0, (MAXC - 1) * C)],
                    out_hbm.at[pl.ds(start * C, (MAXC - 1) * C)])

    @pl.when(n_w == MAXC)
    def _():
        pltpu.sync_copy(scores_v.at[pl.ds((MAXC - 1) * C, C)],
                        out_hbm.at[pl.ds((start + MAXC - 1) * C, C)])


def kernel(h, edge_index):
    u = lax.bitcast_convert_type(h, jnp.uint32)
    rnd = lambda x: x + jnp.uint32(0x7FFF) + ((x >> 16) & jnp.uint32(1))
    lo = rnd(u[:, :D_FEAT // 2]) >> 16
    hi = rnd(u[:, D_FEAT // 2:]) & jnp.uint32(0xFFFF0000)
    h_pk = lax.bitcast_convert_type(lo | hi, jnp.int32)
    call = functools.partial(
        pl.kernel,
        out_type=jax.ShapeDtypeStruct((N_EDGES,), jnp.float32),
        mesh=plsc.VectorSubcoreMesh(core_axis_name="c", subcore_axis_name="s"),
        compiler_params=pltpu.CompilerParams(use_tc_tiling_on_sc=False,
                                             needs_layout_passes=False),
        scratch_types=[
            pltpu.VMEM_SHARED((N_NODES, D_FEAT // 2), jnp.int32),
            pltpu.VMEM((MAXC * C,), jnp.int32),
            pltpu.VMEM((MAXC * C,), jnp.int32),
            pltpu.VMEM((C, D_FEAT // 2), jnp.int32),
            pltpu.VMEM((C, D_FEAT // 2), jnp.int32),
            pltpu.VMEM((C, D_FEAT // 2), jnp.int32),
            pltpu.VMEM((C, D_FEAT // 2), jnp.int32),
            pltpu.VMEM((C, D_FEAT // 2), jnp.int32),
            pltpu.VMEM((C, D_FEAT // 2), jnp.int32),
            pltpu.VMEM((MAXC * C,), jnp.float32),
            pltpu.SemaphoreType.DMA,
            pltpu.SemaphoreType.DMA,
            pltpu.SemaphoreType.DMA,
            pltpu.SemaphoreType.DMA,
            pltpu.SemaphoreType.DMA,
            pltpu.SemaphoreType.DMA,
        ],
    )(_sc_kernel)
    scores = call(h_pk, edge_index)
    return scores.reshape(N_EDGES, 1)
